# baseline (speedup 1.0000x reference)
import jax
import jax.numpy as jnp
from jax.experimental import pallas as pl

DEPTH = 2
DIM = 64
NTOK = 21
TDIM = 16
MDIM = 16
B = 2
N = 256
HMAIN = 256
IBLK = 32

_SELU_L = 1.0507009873554805
_SELU_A = 1.6732632423543772


def _silu(x):
    return x * (1.0 / (1.0 + jnp.exp(-x)))


def _selu(x):
    return _SELU_L * jnp.where(x > 0, x, _SELU_A * (jnp.exp(x) - 1.0))


_N_LAYER_REFS = 22


def _egnn_body(feats_ref, coors_ref, time_ref, *refs):
    (emb_r, wt1_r, bt1_r, wt2_r, bt2_r, wt3_r, bt3_r) = refs[:7]
    layer_refs = refs[7:7 + _N_LAYER_REFS * DEPTH]
    x_out_ref, c_out_ref = refs[7 + _N_LAYER_REFS * DEPTH:]

    emb = emb_r[...]
    for b in range(B):
        fb = feats_ref[b]
        tok_iota = jax.lax.broadcasted_iota(jnp.int32, (N, NTOK), 1)
        onehot = (fb == tok_iota).astype(jnp.float32)
        x = jnp.dot(onehot, emb, preferred_element_type=jnp.float32)

        t = time_ref[b:b + 1]
        t = _selu(jnp.dot(t, wt1_r[...]) + bt1_r[...])
        t = _selu(jnp.dot(t, wt2_r[...]) + bt2_r[...])
        t = jnp.dot(t, wt3_r[...]) + bt3_r[...]
        x = x + t

        c = coors_ref[b]

        for l in range(DEPTH):
            (w1am_r, w1ae_r, w1bm_r, w1be_r, w1dm_r, w1de_r, b1m_r, b1e_r,
             w2m_r, w2e_r, b2_r, wc1_r, bc1_r, wc2_r, bc2_r, lng_r, lnb_r,
             wn1a_r, wn1b_r, bn1_r, wn2_r, bn2_r) = \
                layer_refs[_N_LAYER_REFS * l:_N_LAYER_REFS * (l + 1)]

            a_m = jnp.dot(x, w1am_r[...], preferred_element_type=jnp.float32)
            b_m = jnp.dot(x, w1bm_r[...],
                          preferred_element_type=jnp.float32) + b1m_r[...]
            a_e = jnp.dot(x, w1ae_r[...], preferred_element_type=jnp.float32)
            b_e = jnp.dot(x, w1be_r[...],
                          preferred_element_type=jnp.float32) + b1e_r[...]

            b_m3 = b_m[None, :, :]
            w1dm3 = w1dm_r[...][None, :, :]
            ccols = [c[:, k:k + 1] for k in range(3)]
            cj3 = [col.reshape(1, N, 1) for col in ccols]
            ae3 = [a_e[:, k:k + 1] for k in range(2)]
            be3 = [b_e[:, k:k + 1].reshape(1, N, 1) for k in range(2)]
            w2e = w2e_r[...]

            x_blocks = []
            c_blocks = []
            for ib in range(N // IBLK):
                s = ib * IBLK
                ci3 = [col[s:s + IBLK].reshape(IBLK, 1, 1) for col in ccols]
                rel3 = [ci3[k] - cj3[k] for k in range(3)]
                dist3 = (rel3[0] * rel3[0] + rel3[1] * rel3[1]
                         + rel3[2] * rel3[2])

                a_blk3 = a_m[s:s + IBLK][:, None, :]
                pre = a_blk3 + b_m3 + dist3 * w1dm3
                h = _silu(pre).reshape(IBLK * N, HMAIN)
                m = jnp.dot(h, w2m_r[...],
                            preferred_element_type=jnp.float32) + b2_r[...]
                for k in range(2):
                    w1de_k = w1de_r[...][0:1, k:k + 1].reshape(1, 1, 1)
                    he = _silu(ae3[k][s:s + IBLK].reshape(IBLK, 1, 1)
                               + be3[k] + dist3 * w1de_k)
                    m = m + he.reshape(IBLK * N, 1) * w2e[k:k + 1, :]
                m = _silu(m)
                cwh = _silu(jnp.dot(m, wc1_r[...],
                                    preferred_element_type=jnp.float32)
                            + bc1_r[...])
                cw = (jnp.dot(cwh, wc2_r[...],
                              preferred_element_type=jnp.float32)
                      + bc2_r[...])

                m_i = jnp.sum(m.reshape(IBLK, N, MDIM), axis=1)

                cw3 = cw.reshape(IBLK, N, 1)
                dxyz = [jnp.sum(cw3 * rel3[k], axis=1) for k in range(3)]
                c_blocks.append(c[s:s + IBLK]
                                + jnp.concatenate(dxyz, axis=1))

                xi = x[s:s + IBLK]
                mu = jnp.mean(xi, axis=-1, keepdims=True)
                var = jnp.mean((xi - mu) ** 2, axis=-1, keepdims=True)
                normed = (xi - mu) / jnp.sqrt(var + 1e-5) * lng_r[...] \
                    + lnb_r[...]
                h2 = _silu(jnp.dot(normed, wn1a_r[...],
                                   preferred_element_type=jnp.float32)
                           + jnp.dot(m_i, wn1b_r[...],
                                     preferred_element_type=jnp.float32)
                           + bn1_r[...])
                x_blocks.append(jnp.dot(h2, wn2_r[...],
                                        preferred_element_type=jnp.float32)
                                + xi)

            x = jnp.concatenate(x_blocks, axis=0) + bn2_r[...]
            c = jnp.concatenate(c_blocks, axis=0)

        x_out_ref[b] = x
        c_out_ref[b] = c


@jax.jit
def kernel(feats, coors, time, params):
    feats_i = feats.astype(jnp.int32).reshape(B, N, 1)
    coors_f = coors.astype(jnp.float32)
    time_f = time.astype(jnp.float32).reshape(B, 1)

    def lin(p):
        W, bb = p
        return W, bb.reshape(1, -1)

    args = [feats_i, coors_f, time_f, params['token_emb']]
    for name in ('t1', 't2', 't3'):
        W, bb = lin(params[name])
        args += [W, bb]
    for lp in params['layers']:
        W1, b1 = lin(lp['e1'])
        w1a, w1b, w1d = W1[:DIM], W1[DIM:2 * DIM], W1[2 * DIM:2 * DIM + 1]
        W2, b2 = lin(lp['e2'])
        Wc1, bc1 = lin(lp['c1'])
        Wc2, bc2 = lin(lp['c2'])
        Wn1, bn1 = lin(lp['n1'])
        wn1a, wn1b = Wn1[:DIM], Wn1[DIM:]
        Wn2, bn2 = lin(lp['n2'])
        args += [w1a[:, :HMAIN], w1a[:, HMAIN:], w1b[:, :HMAIN],
                 w1b[:, HMAIN:], w1d[:, :HMAIN], w1d[:, HMAIN:],
                 b1[:, :HMAIN], b1[:, HMAIN:],
                 W2[:HMAIN], W2[HMAIN:], b2, Wc1, bc1, Wc2, bc2,
                 lp['ln_g'].reshape(1, DIM), lp['ln_b'].reshape(1, DIM),
                 wn1a, wn1b, bn1, Wn2, bn2]

    out_shape = (jax.ShapeDtypeStruct((B, N, DIM), jnp.float32),
                 jax.ShapeDtypeStruct((B, N, 3), jnp.float32))
    x_out, c_out = pl.pallas_call(
        _egnn_body,
        out_shape=out_shape,
    )(*args)
    return (x_out, c_out)


# speedup vs baseline: 1.0294x; 1.0294x over previous
import jax
import jax.numpy as jnp
from jax.experimental import pallas as pl

DEPTH = 2
DIM = 64
NTOK = 21
TDIM = 16
MDIM = 16
B = 2
N = 256
HMAIN = 256
HIDE = 2 * (2 * DIM + 1)
IBLK = 32

_SELU_L = 1.0507009873554805
_SELU_A = 1.6732632423543772


def _silu(x):
    return x * (1.0 / (1.0 + jnp.exp(-x)))


def _selu(x):
    return _SELU_L * jnp.where(x > 0, x, _SELU_A * (jnp.exp(x) - 1.0))


_N_LAYER_REFS = 14


def _egnn_body(feats_ref, coors_ref, time_ref, *refs):
    (emb_r, wt1_r, bt1_r, wt2_r, bt2_r, wt3_r, bt3_r) = refs[:7]
    layer_refs = refs[7:7 + _N_LAYER_REFS * DEPTH]
    x_out_ref, c_out_ref = refs[7 + _N_LAYER_REFS * DEPTH:]

    emb = emb_r[...]
    for b in range(B):
        fb = feats_ref[b]
        tok_iota = jax.lax.broadcasted_iota(jnp.int32, (N, NTOK), 1)
        onehot = (fb == tok_iota).astype(jnp.float32)
        x = jnp.dot(onehot, emb, preferred_element_type=jnp.float32)

        t = time_ref[b:b + 1]
        t = _selu(jnp.dot(t, wt1_r[...]) + bt1_r[...])
        t = _selu(jnp.dot(t, wt2_r[...]) + bt2_r[...])
        t = jnp.dot(t, wt3_r[...]) + bt3_r[...]
        x = x + t

        c = coors_ref[b]

        for l in range(DEPTH):
            (w1_r, b1_r, w2_r, b2_r, wc1_r, bc1_r, wc2_r, bc2_r, lng_r,
             lnb_r, wn1_r, bn1_r, wn2_r, bn2_r) = \
                layer_refs[_N_LAYER_REFS * l:_N_LAYER_REFS * (l + 1)]

            w1am = w1_r[0:DIM, 0:HMAIN]
            w1ae = w1_r[0:DIM, HMAIN:HIDE]
            w1bm = w1_r[DIM:2 * DIM, 0:HMAIN]
            w1be = w1_r[DIM:2 * DIM, HMAIN:HIDE]
            w1dm = w1_r[2 * DIM:2 * DIM + 1, 0:HMAIN]
            w1de = w1_r[2 * DIM:2 * DIM + 1, HMAIN:HIDE]
            b1m = b1_r[0:1, 0:HMAIN]
            b1e = b1_r[0:1, HMAIN:HIDE]
            w2m = w2_r[0:HMAIN, :]
            w2e = w2_r[HMAIN:HIDE, :]
            wn1a = wn1_r[0:DIM, :]
            wn1b = wn1_r[DIM:DIM + MDIM, :]

            a_m = jnp.dot(x, w1am, preferred_element_type=jnp.float32)
            b_m = jnp.dot(x, w1bm, preferred_element_type=jnp.float32) + b1m
            a_e = jnp.dot(x, w1ae, preferred_element_type=jnp.float32)
            b_e = jnp.dot(x, w1be, preferred_element_type=jnp.float32) + b1e

            b_m3 = b_m[None, :, :]
            w1dm3 = w1dm[None, :, :]
            ccols = [c[:, k:k + 1] for k in range(3)]
            cj3 = [col.reshape(1, N, 1) for col in ccols]
            ae3 = [a_e[:, k:k + 1] for k in range(2)]
            be3 = [b_e[:, k:k + 1].reshape(1, N, 1) for k in range(2)]

            x_blocks = []
            c_blocks = []
            for ib in range(N // IBLK):
                s = ib * IBLK
                ci3 = [col[s:s + IBLK].reshape(IBLK, 1, 1) for col in ccols]
                rel3 = [ci3[k] - cj3[k] for k in range(3)]
                dist3 = (rel3[0] * rel3[0] + rel3[1] * rel3[1]
                         + rel3[2] * rel3[2])

                a_blk3 = a_m[s:s + IBLK][:, None, :]
                pre = a_blk3 + b_m3 + dist3 * w1dm3
                h = _silu(pre).reshape(IBLK * N, HMAIN)
                m = jnp.dot(h, w2m,
                            preferred_element_type=jnp.float32) + b2_r[...]
                for k in range(2):
                    w1de_k = w1de[0:1, k:k + 1].reshape(1, 1, 1)
                    he = _silu(ae3[k][s:s + IBLK].reshape(IBLK, 1, 1)
                               + be3[k] + dist3 * w1de_k)
                    m = m + he.reshape(IBLK * N, 1) * w2e[k:k + 1, :]
                m = _silu(m)
                cwh = _silu(jnp.dot(m, wc1_r[...],
                                    preferred_element_type=jnp.float32)
                            + bc1_r[...])
                cw = (jnp.dot(cwh, wc2_r[...],
                              preferred_element_type=jnp.float32)
                      + bc2_r[...])

                m_i = jnp.sum(m.reshape(IBLK, N, MDIM), axis=1)

                cw3 = cw.reshape(IBLK, N, 1)
                dxyz = [jnp.sum(cw3 * rel3[k], axis=1) for k in range(3)]
                c_blocks.append(c[s:s + IBLK]
                                + jnp.concatenate(dxyz, axis=1))

                xi = x[s:s + IBLK]
                mu = jnp.mean(xi, axis=-1, keepdims=True)
                var = jnp.mean((xi - mu) ** 2, axis=-1, keepdims=True)
                normed = (xi - mu) / jnp.sqrt(var + 1e-5) * lng_r[...] \
                    + lnb_r[...]
                h2 = _silu(jnp.dot(normed, wn1a,
                                   preferred_element_type=jnp.float32)
                           + jnp.dot(m_i, wn1b,
                                     preferred_element_type=jnp.float32)
                           + bn1_r[...])
                x_blocks.append(jnp.dot(h2, wn2_r[...],
                                        preferred_element_type=jnp.float32)
                                + xi)

            x = jnp.concatenate(x_blocks, axis=0) + bn2_r[...]
            c = jnp.concatenate(c_blocks, axis=0)

        x_out_ref[b] = x
        c_out_ref[b] = c


@jax.jit
def kernel(feats, coors, time, params):
    feats_i = feats.astype(jnp.int32).reshape(B, N, 1)
    coors_f = coors.astype(jnp.float32)
    time_f = time.astype(jnp.float32).reshape(B, 1)

    def lin(p):
        W, bb = p
        return W, bb.reshape(1, -1)

    args = [feats_i, coors_f, time_f, params['token_emb']]
    for name in ('t1', 't2', 't3'):
        W, bb = lin(params[name])
        args += [W, bb]
    for lp in params['layers']:
        W1, b1 = lin(lp['e1'])
        W2, b2 = lin(lp['e2'])
        Wc1, bc1 = lin(lp['c1'])
        Wc2, bc2 = lin(lp['c2'])
        Wn1, bn1 = lin(lp['n1'])
        Wn2, bn2 = lin(lp['n2'])
        args += [W1, b1, W2, b2, Wc1, bc1, Wc2, bc2,
                 lp['ln_g'].reshape(1, DIM), lp['ln_b'].reshape(1, DIM),
                 Wn1, bn1, Wn2, bn2]

    out_shape = (jax.ShapeDtypeStruct((B, N, DIM), jnp.float32),
                 jax.ShapeDtypeStruct((B, N, 3), jnp.float32))
    x_out, c_out = pl.pallas_call(
        _egnn_body,
        out_shape=out_shape,
    )(*args)
    return (x_out, c_out)
